# baseline (speedup 1.0000x reference)
import jax
import jax.numpy as jnp
from jax.experimental import pallas as pl
from jax.experimental.pallas import tpu as pltpu

_N = 2048
_F = 256
_D = 512
_K = 32
_ALPHA = 3.0
_RB = 256
_C = 128
_NC = _N // _C


def _select_pass(v, rem_f):
    m = jnp.max(v, axis=1, keepdims=True)
    li = jax.lax.broadcasted_iota(jnp.int32, (_C, _C), 0)
    lj = jax.lax.broadcasted_iota(jnp.int32, (_C, _C), 1)
    u_incl = jnp.where(li <= lj, 1.0, 0.0).astype(jnp.bfloat16)
    ci = jax.lax.broadcasted_iota(jnp.int32, (_NC, _NC), 0)
    cj = jax.lax.broadcasted_iota(jnp.int32, (_NC, _NC), 1)
    u_strict = jnp.where(ci < cj, 1.0, 0.0).astype(jnp.bfloat16)
    dn = (((1,), (0,)), ((), ()))
    pjs = []
    for j in range(_NC):
        ej = v[:, j * _C:(j + 1) * _C] == m
        eqf = jnp.where(ej, 1.0, 0.0).astype(jnp.bfloat16)
        pjs.append(jax.lax.dot_general(eqf, u_incl, dn,
                                       preferred_element_type=jnp.float32))
    ct = jnp.concatenate([pj[:, _C - 1:_C] for pj in pjs], axis=1)
    cpc = jax.lax.dot_general(ct.astype(jnp.bfloat16), u_strict, dn,
                              preferred_element_type=jnp.float32)
    cnt = cpc[:, _NC - 1:_NC] + ct[:, _NC - 1:_NC]
    thrs = [rem_f - cpc[:, j:j + 1] for j in range(_NC)]
    return m, thrs, pjs, cnt


_S = _N // _RB


def _graph_kernel(x_ref, w1_ref, b1_ref, w2_ref, b2_ref, out_ref,
                  nv1_s, nv2_s, adjbuf, vbuf, rem_ref):
    f = pl.program_id(0)
    produce = f < pl.num_programs(0) - 1
    ip = jax.lax.rem(f, _S)

    @pl.when(produce & (ip == 0))
    def _():
        x = x_ref[0]
        dnf = (((1,), (1,)), ((), ()))
        h1 = jax.lax.dot_general(x, w1_ref[...], dnf,
                                 preferred_element_type=jnp.float32)
        nv1_s[...] = jnp.tanh(_ALPHA * (h1 + b1_ref[...]))
        h2 = jax.lax.dot_general(x, w2_ref[...], dnf,
                                 preferred_element_type=jnp.float32)
        nv2_s[...] = jnp.tanh(_ALPHA * (h2 + b2_ref[...]))

    @pl.when(produce)
    def _():
        r0 = ip * _RB
        nv1 = nv1_s[...]
        nv2 = nv2_s[...]
        nv1r = nv1_s[pl.ds(r0, _RB), :]
        nv2r = nv2_s[pl.ds(r0, _RB), :]
        dnd = (((1,), (1,)), ((), ()))
        a = jax.lax.dot_general(nv1r, nv2, dnd,
                                preferred_element_type=jnp.float32)
        a -= jax.lax.dot_general(nv2r, nv1, dnd,
                                 preferred_element_type=jnp.float32)
        adjbuf[jax.lax.rem(f, 2)] = jnp.maximum(jnp.tanh(_ALPHA * a), 0.0)

    @pl.when(f > 0)
    def _():
        adj = adjbuf[jax.lax.rem(f + 1, 2)]
        m, thrs, pjs, cnt = _select_pass(
            adj, jnp.full((_RB, 1), float(_K), jnp.float32))
        for j in range(_NC):
            sl = slice(j * _C, (j + 1) * _C)
            aj = adj[:, sl]
            out_ref[0, :, sl] = jnp.where((aj == m) & (pjs[j] <= thrs[j]),
                                          aj, 0.0)
        rem1 = _K - jnp.minimum(cnt.astype(jnp.int32), _K)
        tot1 = jnp.sum(rem1)

        @pl.when(tot1 > 0)
        def _():
            for j in range(_NC):
                sl = slice(j * _C, (j + 1) * _C)
                aj = adj[:, sl]
                vbuf[:, sl] = jnp.where((aj == m) & (pjs[j] <= thrs[j]),
                                        -1.0, aj)
            rem_ref[...] = rem1

        def cond(tot):
            return tot > 0

        def body(tot):
            v = vbuf[...]
            remv = rem_ref[...]
            m2, thrs2, pjs2, cnt2 = _select_pass(v, remv.astype(jnp.float32))
            for j in range(_NC):
                sl = slice(j * _C, (j + 1) * _C)
                vj = v[:, sl]
                take2 = (vj == m2) & (pjs2[j] <= thrs2[j])
                out_ref[0, :, sl] = jnp.where(take2, vj, out_ref[0, :, sl])
            rem_new = remv - jnp.minimum(cnt2.astype(jnp.int32), remv)
            tot_new = jnp.sum(rem_new)

            @pl.when(tot_new > 0)
            def _():
                for j in range(_NC):
                    sl = slice(j * _C, (j + 1) * _C)
                    vj = v[:, sl]
                    vbuf[:, sl] = jnp.where(
                        (vj == m2) & (pjs2[j] <= thrs2[j]), -1.0, vj)
                rem_ref[...] = rem_new

            return tot_new

        jax.lax.while_loop(cond, body, tot1)


def kernel(X, W1, b1, W2, b2):
    B = X.shape[0]
    b1r = b1.reshape(1, _D)
    b2r = b2.reshape(1, _D)

    def x_map(f):
        return (jnp.minimum(f // _S, B - 1), 0, 0)

    def out_map(f):
        fm = jnp.maximum(f, 1) - 1
        return (fm // _S, jax.lax.rem(fm, _S), 0)

    adj = pl.pallas_call(
        _graph_kernel,
        grid=(B * _S + 1,),
        in_specs=[
            pl.BlockSpec((1, _N, _F), x_map),
            pl.BlockSpec((_D, _F), lambda f: (0, 0)),
            pl.BlockSpec((1, _D), lambda f: (0, 0)),
            pl.BlockSpec((_D, _F), lambda f: (0, 0)),
            pl.BlockSpec((1, _D), lambda f: (0, 0)),
        ],
        out_specs=pl.BlockSpec((1, _RB, _N), out_map),
        out_shape=jax.ShapeDtypeStruct((B, _N, _N), jnp.float32),
        scratch_shapes=[
            pltpu.VMEM((_N, _D), jnp.float32),
            pltpu.VMEM((_N, _D), jnp.float32),
            pltpu.VMEM((2, _RB, _N), jnp.float32),
            pltpu.VMEM((_RB, _N), jnp.float32),
            pltpu.VMEM((_RB, 1), jnp.int32),
        ],
    )(X, W1, b1r, W2, b2r)

    return adj


# speedup vs baseline: 1.1081x; 1.1081x over previous
import jax
import jax.numpy as jnp
from jax.experimental import pallas as pl
from jax.experimental.pallas import tpu as pltpu

_N = 2048
_F = 256
_D = 512
_K = 32
_ALPHA = 3.0
_RB = 256
_C = 128
_NC = _N // _C


def _select_pass(v, rem_f):
    m = jnp.max(v, axis=1, keepdims=True)
    li = jax.lax.broadcasted_iota(jnp.int32, (_C, _C), 0)
    lj = jax.lax.broadcasted_iota(jnp.int32, (_C, _C), 1)
    u_incl = jnp.where(li <= lj, 1.0, 0.0).astype(jnp.bfloat16)
    ci = jax.lax.broadcasted_iota(jnp.int32, (_NC, _NC), 0)
    cj = jax.lax.broadcasted_iota(jnp.int32, (_NC, _NC), 1)
    u_strict = jnp.where(ci < cj, 1.0, 0.0).astype(jnp.bfloat16)
    dn = (((1,), (0,)), ((), ()))
    pjs = []
    for j in range(_NC):
        ej = v[:, j * _C:(j + 1) * _C] == m
        eqf = jnp.where(ej, 1.0, 0.0).astype(jnp.bfloat16)
        pjs.append(jax.lax.dot_general(eqf, u_incl, dn,
                                       preferred_element_type=jnp.float32))
    ct = jnp.concatenate([pj[:, _C - 1:_C] for pj in pjs], axis=1)
    cpc = jax.lax.dot_general(ct.astype(jnp.bfloat16), u_strict, dn,
                              preferred_element_type=jnp.float32)
    cnt = cpc[:, _NC - 1:_NC] + ct[:, _NC - 1:_NC]
    thrs = [rem_f - cpc[:, j:j + 1] for j in range(_NC)]
    return m, thrs, pjs, cnt


_S = _N // _RB


def _graph_kernel(x_ref, w1_ref, b1_ref, w2_ref, b2_ref, out_ref,
                  nv1_s, nv2_s, adjbuf, vbuf, rem_ref):
    f = pl.program_id(0)
    fp = jnp.minimum(f, pl.num_programs(0) - 2)
    ip = jax.lax.rem(fp, _S)

    @pl.when(f == 0)
    def _():
        adjbuf[1] = jnp.zeros((_RB, _N), jnp.float32)

    @pl.when((ip == 0) & (f < pl.num_programs(0) - 1))
    def _():
        x = x_ref[0]
        dnf = (((1,), (1,)), ((), ()))
        h1 = jax.lax.dot_general(x, w1_ref[...], dnf,
                                 preferred_element_type=jnp.float32)
        nv1_s[...] = jnp.tanh(_ALPHA * (h1 + b1_ref[...]))
        h2 = jax.lax.dot_general(x, w2_ref[...], dnf,
                                 preferred_element_type=jnp.float32)
        nv2_s[...] = jnp.tanh(_ALPHA * (h2 + b2_ref[...]))

    adj = adjbuf[jax.lax.rem(f + 1, 2)]
    m, thrs, pjs, cnt = _select_pass(
        adj, jnp.full((_RB, 1), float(_K), jnp.float32))
    for j in range(_NC):
        sl = slice(j * _C, (j + 1) * _C)
        aj = adj[:, sl]
        out_ref[0, :, sl] = jnp.where((aj == m) & (pjs[j] <= thrs[j]),
                                      aj, 0.0)
    rem1 = _K - jnp.minimum(cnt.astype(jnp.int32), _K)
    tot1 = jnp.sum(rem1)

    r0 = ip * _RB
    nv1 = nv1_s[...]
    nv2 = nv2_s[...]
    nv1r = nv1_s[pl.ds(r0, _RB), :]
    nv2r = nv2_s[pl.ds(r0, _RB), :]
    dnd = (((1,), (1,)), ((), ()))
    a = jax.lax.dot_general(nv1r, nv2, dnd,
                            preferred_element_type=jnp.float32)
    a -= jax.lax.dot_general(nv2r, nv1, dnd,
                             preferred_element_type=jnp.float32)
    adjbuf[jax.lax.rem(f, 2)] = jnp.maximum(jnp.tanh(_ALPHA * a), 0.0)

    @pl.when(tot1 > 0)
    def _():
        for j in range(_NC):
            sl = slice(j * _C, (j + 1) * _C)
            aj = adj[:, sl]
            vbuf[:, sl] = jnp.where((aj == m) & (pjs[j] <= thrs[j]),
                                    -1.0, aj)
        rem_ref[...] = rem1

    def cond(carry):
        tot, it = carry
        return (tot > 0) & (it < _K)

    def body(carry):
        tot, it = carry
        v = vbuf[...]
        remv = rem_ref[...]
        m2, thrs2, pjs2, cnt2 = _select_pass(v, remv.astype(jnp.float32))
        for j in range(_NC):
            sl = slice(j * _C, (j + 1) * _C)
            vj = v[:, sl]
            take2 = (vj == m2) & (pjs2[j] <= thrs2[j])
            out_ref[0, :, sl] = jnp.where(take2, vj, out_ref[0, :, sl])
        rem_new = remv - jnp.minimum(cnt2.astype(jnp.int32), remv)
        tot_new = jnp.sum(rem_new)

        @pl.when(tot_new > 0)
        def _():
            for j in range(_NC):
                sl = slice(j * _C, (j + 1) * _C)
                vj = v[:, sl]
                vbuf[:, sl] = jnp.where(
                    (vj == m2) & (pjs2[j] <= thrs2[j]), -1.0, vj)
            rem_ref[...] = rem_new

        return (tot_new, it + 1)

    jax.lax.while_loop(cond, body, (tot1, jnp.int32(0)))


def kernel(X, W1, b1, W2, b2):
    B = X.shape[0]
    b1r = b1.reshape(1, _D)
    b2r = b2.reshape(1, _D)

    def x_map(f):
        return (jnp.minimum(f // _S, B - 1), 0, 0)

    def out_map(f):
        fm = jnp.maximum(f, 1) - 1
        return (fm // _S, jax.lax.rem(fm, _S), 0)

    adj = pl.pallas_call(
        _graph_kernel,
        grid=(B * _S + 1,),
        in_specs=[
            pl.BlockSpec((1, _N, _F), x_map),
            pl.BlockSpec((_D, _F), lambda f: (0, 0)),
            pl.BlockSpec((1, _D), lambda f: (0, 0)),
            pl.BlockSpec((_D, _F), lambda f: (0, 0)),
            pl.BlockSpec((1, _D), lambda f: (0, 0)),
        ],
        out_specs=pl.BlockSpec((1, _RB, _N), out_map),
        out_shape=jax.ShapeDtypeStruct((B, _N, _N), jnp.float32),
        scratch_shapes=[
            pltpu.VMEM((_N, _D), jnp.float32),
            pltpu.VMEM((_N, _D), jnp.float32),
            pltpu.VMEM((2, _RB, _N), jnp.float32),
            pltpu.VMEM((_RB, _N), jnp.float32),
            pltpu.VMEM((_RB, 1), jnp.int32),
        ],
    )(X, W1, b1r, W2, b2r)

    return adj
